# baseline (device time: 10094 ns/iter reference)
import jax
import jax.numpy as jnp
from jax import lax
from jax.experimental import pallas as pl
from jax.experimental.pallas import tpu as pltpu

N_DEV = 4
N_TOK = 256
D_IN = 128
D_OUT = 256
N_EXPERTS = 8
E_LOCAL = 2
SR, SC = 2, 128


def kernel(x, router_W, route_idx, expert_W, shared_W):
    def body(x_ref, rw_ref, idx_ref, ew_ref, sw_ref, out_ref,
             commq_ref, comms_ref, send_sems, recv_sems,
             send_sems_s, recv_sems_s):
        my = lax.axis_index("i")

        barrier = pltpu.get_barrier_semaphore()
        for k in range(1, N_DEV):
            pl.semaphore_signal(
                barrier, inc=1,
                device_id=((my + k) % N_DEV,),
                device_id_type=pl.DeviceIdType.MESH,
            )

        xf = x_ref[:, :]
        xb = xf.astype(jnp.bfloat16)

        w_cat = jnp.concatenate(
            [ew_ref[0].astype(jnp.bfloat16),
             ew_ref[1].astype(jnp.bfloat16),
             sw_ref[:, :].astype(jnp.bfloat16)], axis=1)
        y = jnp.dot(xb, w_cat, preferred_element_type=jnp.float32)

        scores = jnp.dot(xf, rw_ref[:, :], preferred_element_type=jnp.float32)
        scores = scores - jnp.max(scores, axis=-1, keepdims=True)
        p = jnp.exp(scores)
        probs = p / jnp.sum(p, axis=-1, keepdims=True)

        idx = idx_ref[:, :]
        cols = lax.broadcasted_iota(jnp.int32, (N_TOK, N_EXPERTS), 1)
        chosen = jnp.sum(jnp.where(cols == idx, probs, 0.0),
                         axis=-1, keepdims=True)
        c0 = jnp.where(idx == E_LOCAL * my, chosen, 0.0)
        c1 = jnp.where(idx == E_LOCAL * my + 1, chosen, 0.0)

        partial = c0 * y[:, :D_OUT] + c1 * y[:, D_OUT:2 * D_OUT]

        p3 = partial.reshape(SR, SC, D_OUT)
        amax = jnp.max(jnp.abs(p3), axis=2, keepdims=True)
        inv = jnp.where(amax > 0, 127.0 / amax, 0.0)
        commq_ref[0] = jnp.round(p3 * inv).astype(jnp.int8).reshape(
            N_TOK, D_OUT)
        comms_ref[0] = (amax * (1.0 / 127.0)).reshape(SR, SC)

        pl.semaphore_wait(barrier, N_DEV - 1)

        rdmas = []
        for k in range(1, N_DEV):
            dest = ((my + k) % N_DEV,)
            rq = pltpu.make_async_remote_copy(
                src_ref=commq_ref.at[0],
                dst_ref=commq_ref.at[k],
                send_sem=send_sems.at[k - 1],
                recv_sem=recv_sems.at[k - 1],
                device_id=dest,
                device_id_type=pl.DeviceIdType.MESH,
            )
            rs = pltpu.make_async_remote_copy(
                src_ref=comms_ref.at[0],
                dst_ref=comms_ref.at[k],
                send_sem=send_sems_s.at[k - 1],
                recv_sem=recv_sems_s.at[k - 1],
                device_id=dest,
                device_id_type=pl.DeviceIdType.MESH,
            )
            rq.start()
            rs.start()
            rdmas.append((rq, rs))

        acc = partial + y[:, 2 * D_OUT:]

        for k in range(1, N_DEV):
            rq, rs = rdmas[k - 1]
            rq.wait()
            rs.wait()
            qk = commq_ref[k].reshape(SR, SC, D_OUT).astype(jnp.float32)
            sk = comms_ref[k].reshape(SR, SC, 1)
            acc = acc + (qk * sk).reshape(N_TOK, D_OUT)
        out_ref[:, :] = acc

    return pl.pallas_call(
        body,
        out_shape=jax.ShapeDtypeStruct((N_TOK, D_OUT), jnp.float32),
        in_specs=[pl.BlockSpec(memory_space=pltpu.VMEM)] * 5,
        out_specs=pl.BlockSpec(memory_space=pltpu.VMEM),
        scratch_shapes=[
            pltpu.VMEM((N_DEV, N_TOK, D_OUT), jnp.int8),
            pltpu.VMEM((N_DEV, SR, SC), jnp.float32),
            pltpu.SemaphoreType.DMA((N_DEV - 1,)),
            pltpu.SemaphoreType.DMA((N_DEV - 1,)),
            pltpu.SemaphoreType.DMA((N_DEV - 1,)),
            pltpu.SemaphoreType.DMA((N_DEV - 1,)),
        ],
        compiler_params=pltpu.CompilerParams(collective_id=0),
    )(x, router_W, route_idx, expert_W, shared_W)


# device time: 10066 ns/iter; 1.0028x vs baseline; 1.0028x over previous
import jax
import jax.numpy as jnp
from jax import lax
from jax.experimental import pallas as pl
from jax.experimental.pallas import tpu as pltpu

N_DEV = 4
N_TOK = 256
D_IN = 128
D_OUT = 256
N_EXPERTS = 8
E_LOCAL = 2
SR, SC = 2, 128


def kernel(x, router_W, route_idx, expert_W, shared_W):
    def body(x_ref, rw_ref, idx_ref, ew_ref, sw_ref, out_ref,
             commq_ref, comms_ref, send_sems, recv_sems,
             send_sems_s, recv_sems_s):
        my = lax.axis_index("i")

        barrier = pltpu.get_barrier_semaphore()
        for k in range(1, N_DEV):
            pl.semaphore_signal(
                barrier, inc=1,
                device_id=((my + k) % N_DEV,),
                device_id_type=pl.DeviceIdType.MESH,
            )

        xf = x_ref[:, :]
        xb = xf.astype(jnp.bfloat16)

        w_cat = jnp.concatenate(
            [ew_ref[0].astype(jnp.bfloat16),
             ew_ref[1].astype(jnp.bfloat16),
             sw_ref[:, :].astype(jnp.bfloat16)], axis=1)
        y = jnp.dot(xb, w_cat, preferred_element_type=jnp.float32)

        scores = jnp.dot(xf, rw_ref[:, :], preferred_element_type=jnp.float32)
        scores = scores - jnp.max(scores, axis=-1, keepdims=True)
        p = jnp.exp(scores)
        probs = p / jnp.sum(p, axis=-1, keepdims=True)

        idx = idx_ref[:, :]
        cols = lax.broadcasted_iota(jnp.int32, (N_TOK, N_EXPERTS), 1)
        chosen = jnp.sum(jnp.where(cols == idx, probs, 0.0),
                         axis=-1, keepdims=True)
        c0 = jnp.where(idx == E_LOCAL * my, chosen, 0.0)
        c1 = jnp.where(idx == E_LOCAL * my + 1, chosen, 0.0)

        partial = c0 * y[:, :D_OUT] + c1 * y[:, D_OUT:2 * D_OUT]

        p3 = partial.reshape(SR, SC, D_OUT)
        amax = jnp.max(jnp.abs(p3), axis=2, keepdims=True)
        inv = jnp.where(amax > 0, 127.0 / amax, 0.0)
        commq_ref[0] = jnp.round(p3 * inv).astype(jnp.int8).reshape(
            N_TOK, D_OUT)
        comms_ref[0] = (amax * (1.0 / 127.0)).reshape(SR, SC)

        pl.semaphore_wait(barrier, N_DEV - 1)

        rdmas = []
        for k in range(1, N_DEV):
            dest = ((my + k) % N_DEV,)
            rq = pltpu.make_async_remote_copy(
                src_ref=commq_ref.at[0],
                dst_ref=commq_ref.at[k],
                send_sem=send_sems.at[k - 1],
                recv_sem=recv_sems.at[k - 1],
                device_id=dest,
                device_id_type=pl.DeviceIdType.MESH,
            )
            rs = pltpu.make_async_remote_copy(
                src_ref=comms_ref.at[0],
                dst_ref=comms_ref.at[k],
                send_sem=send_sems_s.at[k - 1],
                recv_sem=recv_sems_s.at[k - 1],
                device_id=dest,
                device_id_type=pl.DeviceIdType.MESH,
            )
            rq.start()
            rs.start()
            rdmas.append((rq, rs))

        acc = partial + y[:, 2 * D_OUT:]

        for rq, rs in rdmas:
            rq.wait()
            rs.wait()
        qsum = (commq_ref[1].astype(jnp.int16) + commq_ref[2].astype(jnp.int16)
                + commq_ref[3].astype(jnp.int16))
        ssum = comms_ref[1] + comms_ref[2] + comms_ref[3]
        deq = qsum.reshape(SR, SC, D_OUT).astype(jnp.float32) * ssum.reshape(
            SR, SC, 1)
        out_ref[:, :] = acc + deq.reshape(N_TOK, D_OUT)

    return pl.pallas_call(
        body,
        out_shape=jax.ShapeDtypeStruct((N_TOK, D_OUT), jnp.float32),
        in_specs=[pl.BlockSpec(memory_space=pltpu.VMEM)] * 5,
        out_specs=pl.BlockSpec(memory_space=pltpu.VMEM),
        scratch_shapes=[
            pltpu.VMEM((N_DEV, N_TOK, D_OUT), jnp.int8),
            pltpu.VMEM((N_DEV, SR, SC), jnp.float32),
            pltpu.SemaphoreType.DMA((N_DEV - 1,)),
            pltpu.SemaphoreType.DMA((N_DEV - 1,)),
            pltpu.SemaphoreType.DMA((N_DEV - 1,)),
            pltpu.SemaphoreType.DMA((N_DEV - 1,)),
        ],
        compiler_params=pltpu.CompilerParams(collective_id=0),
    )(x, router_W, route_idx, expert_W, shared_W)
